# baseline (speedup 1.0000x reference)
import functools

import jax
import jax.numpy as jnp
from jax import lax
from jax.experimental import pallas as pl
from jax.experimental.pallas import tpu as pltpu
from jax.experimental.pallas import tpu_sc as plsc

NC = 2
NS = 16
NW = NC * NS
LANES = 16


def _word_context_sum(word_table, flat_ids, seg_ids, B, L, D, GB=128):
    per_w = B // NW
    ids_per_w = per_w * L
    n_b = ids_per_w // GB
    mesh = plsc.VectorSubcoreMesh(core_axis_name="c", subcore_axis_name="s")

    @functools.partial(
        pl.kernel,
        out_type=jax.ShapeDtypeStruct((B, D), jnp.float32),
        mesh=mesh,
        compiler_params=pltpu.CompilerParams(use_tc_tiling_on_sc=False),
        scratch_types=[
            pltpu.VMEM((ids_per_w,), jnp.int32),
            pltpu.VMEM((n_b, GB), jnp.int32),
            pltpu.VMEM((GB, D), jnp.float32),
            pltpu.VMEM((per_w, D), jnp.float32),
            pltpu.VMEM_SHARED((NS * per_w, D), jnp.float32),
        ],
    )
    def k(tbl_hbm, ids_hbm, seg_hbm, out_hbm, ids_v, seg_v, rows_v, zero_v,
          acc_sh):
        c = lax.axis_index("c")
        s_idx = lax.axis_index("s")
        wid = s_idx * NC + c
        base_b = wid * per_w
        local_base = s_idx * per_w
        pltpu.sync_copy(ids_hbm.at[pl.ds(base_b * L, ids_per_w)], ids_v)
        pltpu.sync_copy(seg_hbm, seg_v)

        off = jnp.full((LANES,), local_base, jnp.int32)

        @pl.loop(0, n_b)
        def _(t):
            for j in range(GB // LANES):
                sl = pl.ds(j * LANES, LANES)
                seg_v.at[t, sl][...] = seg_v.at[t, sl][...] + off

        @pl.loop(0, per_w)
        def _(i):
            for j in range(D // LANES):
                zero_v.at[i, pl.ds(j * LANES, LANES)][...] = jnp.zeros(
                    (LANES,), jnp.float32)

        pltpu.sync_copy(zero_v, acc_sh.at[pl.ds(local_base, per_w)])

        @pl.loop(0, n_b)
        def _(t):
            pltpu.sync_copy(
                tbl_hbm.at[ids_v.at[pl.ds(t * GB, GB)]], rows_v)
            pltpu.sync_copy(rows_v, acc_sh.at[seg_v.at[t]], add=True)

        pltpu.sync_copy(acc_sh.at[pl.ds(local_base, per_w)],
                        out_hbm.at[pl.ds(base_b, per_w)])

    return k(word_table, flat_ids, seg_ids)


def _ent_gather(ent_table, flat_ids, N, D, GB=128):
    per_w = N // NW
    n_b = per_w // GB
    mesh = plsc.VectorSubcoreMesh(core_axis_name="c", subcore_axis_name="s")

    @functools.partial(
        pl.kernel,
        out_type=jax.ShapeDtypeStruct((N, D), jnp.float32),
        mesh=mesh,
        compiler_params=pltpu.CompilerParams(use_tc_tiling_on_sc=False),
        scratch_types=[
            pltpu.VMEM((per_w,), jnp.int32),
            pltpu.VMEM((GB, D), jnp.float32),
        ],
    )
    def k(tbl_hbm, ids_hbm, out_hbm, ids_v, rows_v):
        wid = lax.axis_index("s") * NC + lax.axis_index("c")
        base = wid * per_w
        pltpu.sync_copy(ids_hbm.at[pl.ds(base, per_w)], ids_v)

        @pl.loop(0, n_b)
        def _(t):
            pltpu.sync_copy(
                tbl_hbm.at[ids_v.at[pl.ds(t * GB, GB)]], rows_v)
            pltpu.sync_copy(rows_v, out_hbm.at[pl.ds(base + t * GB, GB)])

    return k(ent_table, flat_ids)


def _tc_scores(ctx_sum, Wt_scaled, bias, ent_rows, B, C, D, blk=256):
    grid = (B // blk,)

    def body(m_ref, wt_ref, b_ref, ent_ref, out_ref):
        ctx = jnp.dot(m_ref[...], wt_ref[...],
                      preferred_element_type=jnp.float32) + b_ref[...]
        cand = ent_ref[...].reshape(blk, C, D)
        out_ref[...] = jnp.sum(cand * ctx[:, None, :], axis=-1)

    return pl.pallas_call(
        body,
        grid=grid,
        in_specs=[
            pl.BlockSpec((blk, D), lambda i: (i, 0)),
            pl.BlockSpec((D, D), lambda i: (0, 0)),
            pl.BlockSpec((1, D), lambda i: (0, 0)),
            pl.BlockSpec((blk * C, D), lambda i: (i, 0)),
        ],
        out_specs=pl.BlockSpec((blk, C), lambda i: (i, 0)),
        out_shape=jax.ShapeDtypeStruct((B, C), jnp.float32),
    )(ctx_sum, Wt_scaled, bias, ent_rows)


def kernel(word_table, ent_table, W, b, context_ids, cand_ids):
    B, L = context_ids.shape
    _, C = cand_ids.shape
    D = word_table.shape[1]

    flat_ctx = context_ids.reshape(-1).astype(jnp.int32)
    flat_cand = cand_ids.reshape(-1).astype(jnp.int32)
    ids_per_w = (B // NW) * L
    seg_ids = (jnp.arange(ids_per_w, dtype=jnp.int32) // L).reshape(
        ids_per_w // 128, 128)

    ctx_sum = _word_context_sum(word_table, flat_ctx, seg_ids, B, L, D)
    ent_rows = _ent_gather(ent_table, flat_cand, B * C, D)

    Wt_scaled = (W.T / L).astype(jnp.float32)
    bias = b.reshape(1, D).astype(jnp.float32)
    return _tc_scores(ctx_sum, Wt_scaled, bias, ent_rows, B, C, D)


# speedup vs baseline: 1.0320x; 1.0320x over previous
import functools

import jax
import jax.numpy as jnp
from jax import lax
from jax.experimental import pallas as pl
from jax.experimental.pallas import tpu as pltpu
from jax.experimental.pallas import tpu_sc as plsc

NC = 2
NS = 16
NW = NC * NS
LANES = 16


def _word_context_sum(word_table, flat_ids, seg_ids, B, L, D, GB=128):
    per_w = B // NW
    ids_per_w = per_w * L
    n_b = ids_per_w // GB
    mesh = plsc.VectorSubcoreMesh(core_axis_name="c", subcore_axis_name="s")

    @functools.partial(
        pl.kernel,
        out_type=jax.ShapeDtypeStruct((B, D), jnp.float32),
        mesh=mesh,
        compiler_params=pltpu.CompilerParams(use_tc_tiling_on_sc=False),
        scratch_types=[
            pltpu.VMEM((ids_per_w,), jnp.int32),
            pltpu.VMEM((n_b, GB), jnp.int32),
            pltpu.VMEM((GB, D), jnp.float32),
            pltpu.VMEM((per_w, D), jnp.float32),
            pltpu.VMEM_SHARED((NS * per_w, D), jnp.float32),
        ],
    )
    def k(tbl_hbm, ids_hbm, seg_hbm, out_hbm, ids_v, seg_v, rows_v, zero_v,
          acc_sh):
        c = lax.axis_index("c")
        s_idx = lax.axis_index("s")
        wid = s_idx * NC + c
        base_b = wid * per_w
        local_base = s_idx * per_w
        pltpu.sync_copy(ids_hbm.at[pl.ds(base_b * L, ids_per_w)], ids_v)
        pltpu.sync_copy(seg_hbm, seg_v)

        off = jnp.full((LANES,), local_base, jnp.int32)

        @pl.loop(0, n_b)
        def _(t):
            for j in range(GB // LANES):
                sl = pl.ds(j * LANES, LANES)
                seg_v.at[t, sl][...] = seg_v.at[t, sl][...] + off

        @pl.loop(0, per_w)
        def _(i):
            for j in range(D // LANES):
                zero_v.at[i, pl.ds(j * LANES, LANES)][...] = jnp.zeros(
                    (LANES,), jnp.float32)

        pltpu.sync_copy(zero_v, acc_sh.at[pl.ds(local_base, per_w)])

        @pl.loop(0, n_b)
        def _(t):
            pltpu.sync_copy(
                tbl_hbm.at[ids_v.at[pl.ds(t * GB, GB)]], rows_v)
            pltpu.sync_copy(rows_v, acc_sh.at[seg_v.at[t]], add=True)

        pltpu.sync_copy(acc_sh.at[pl.ds(local_base, per_w)],
                        out_hbm.at[pl.ds(base_b, per_w)])

    return k(word_table, flat_ids, seg_ids)


def _ent_gather(ent_table, flat_ids, N, D, GB=128):
    per_w = N // NW
    n_b = per_w // GB
    mesh = plsc.VectorSubcoreMesh(core_axis_name="c", subcore_axis_name="s")

    @functools.partial(
        pl.kernel,
        out_type=jax.ShapeDtypeStruct((N, D), jnp.float32),
        mesh=mesh,
        compiler_params=pltpu.CompilerParams(use_tc_tiling_on_sc=False),
        scratch_types=[
            pltpu.VMEM((per_w,), jnp.int32),
            pltpu.VMEM((GB, D), jnp.float32),
        ],
    )
    def k(tbl_hbm, ids_hbm, out_hbm, ids_v, rows_v):
        wid = lax.axis_index("s") * NC + lax.axis_index("c")
        base = wid * per_w
        pltpu.sync_copy(ids_hbm.at[pl.ds(base, per_w)], ids_v)

        @pl.loop(0, n_b)
        def _(t):
            pltpu.sync_copy(
                tbl_hbm.at[ids_v.at[pl.ds(t * GB, GB)]], rows_v)
            pltpu.sync_copy(rows_v, out_hbm.at[pl.ds(base + t * GB, GB)])

    return k(ent_table, flat_ids)


def _tc_scores(ctx_sum, Wt_scaled, bias, ent_rows, B, C, D, blk=256):
    grid = (B // blk,)

    def body(m_ref, wt_ref, b_ref, ent_ref, out_ref):
        ctx = jnp.dot(m_ref[...], wt_ref[...],
                      preferred_element_type=jnp.float32) + b_ref[...]
        cand = ent_ref[...].reshape(blk, C, D)
        out_ref[...] = jnp.sum(cand * ctx[:, None, :], axis=-1)

    return pl.pallas_call(
        body,
        grid=grid,
        in_specs=[
            pl.BlockSpec((blk, D), lambda i: (i, 0)),
            pl.BlockSpec((D, D), lambda i: (0, 0)),
            pl.BlockSpec((1, D), lambda i: (0, 0)),
            pl.BlockSpec((blk, C * D), lambda i: (i, 0)),
        ],
        out_specs=pl.BlockSpec((blk, C), lambda i: (i, 0)),
        out_shape=jax.ShapeDtypeStruct((B, C), jnp.float32),
    )(ctx_sum, Wt_scaled, bias, ent_rows.reshape(B, C * D))


def kernel(word_table, ent_table, W, b, context_ids, cand_ids):
    B, L = context_ids.shape
    _, C = cand_ids.shape
    D = word_table.shape[1]

    flat_ctx = context_ids.reshape(-1).astype(jnp.int32)
    flat_cand = cand_ids.reshape(-1).astype(jnp.int32)
    ids_per_w = (B // NW) * L
    seg_ids = (jnp.arange(ids_per_w, dtype=jnp.int32) // L).reshape(
        ids_per_w // 128, 128)

    ctx_sum = _word_context_sum(word_table, flat_ctx, seg_ids, B, L, D)
    ent_rows = _ent_gather(ent_table, flat_cand, B * C, D)

    Wt_scaled = (W.T / L).astype(jnp.float32)
    bias = b.reshape(1, D).astype(jnp.float32)
    return _tc_scores(ctx_sum, Wt_scaled, bias, ent_rows, B, C, D)
